# baseline (speedup 1.0000x reference)
import jax
import jax.numpy as jnp
import numpy as np
from jax import lax
from jax.experimental import pallas as pl
from jax.experimental.pallas import tpu as pltpu
from jax.experimental.pallas import tpu_sc as plsc

N_BATCH = 128
ROW = 30
N_ROWS = 128 * 14 * 14
NC, NS, L = 2, 16, 16
N_W = NC * NS
ROWS_PER_W = N_ROWS // N_W
CHUNKS = ROWS_PER_W // L
SLAB = ROWS_PER_W * ROW
N_SEC = 7
SEC_CHUNKS = CHUNKS // N_SEC
SEC = SEC_CHUNKS * L * ROW

INV14 = np.float32(1.0 / 14.0)


def _sqrt(x):
    i = lax.bitcast_convert_type(x, jnp.int32)
    i = 0x5F3759DF - jnp.right_shift(i, 1)
    y = lax.bitcast_convert_type(i, jnp.float32)
    for _ in range(3):
        y = y * (1.5 - 0.5 * x * y * y)
    return x * y


def _corners(x, y, w, h):
    cx = x * INV14
    cy = y * INV14
    return cx - 0.5 * w, cy - 0.5 * h, cx + 0.5 * w, cy + 0.5 * h


def _iou(px, py, pw, ph, tx0, ty0, tx1, ty1, ta):
    ax0, ay0, ax1, ay1 = _corners(px, py, pw, ph)
    iw = jnp.maximum(jnp.minimum(ax1, tx1) - jnp.maximum(ax0, tx0), 0.0)
    ih = jnp.maximum(jnp.minimum(ay1, ty1) - jnp.maximum(ay0, ty0), 0.0)
    inter = iw * ih
    a1 = (ax1 - ax0) * (ay1 - ay0)
    return inter / (a1 + ta - inter)


def _body(pred_hbm, targ_hbm, out_hbm, pred_v0, pred_v1, targ_v0, targ_v1,
          stage_v, psem0, psem1, tsem0, tsem1):
    cc = lax.axis_index("c")
    ss = lax.axis_index("s")
    wid = cc * NS + ss
    base = wid * SLAB
    psem = (psem0, psem1)
    tsem = (tsem0, tsem1)
    pbuf = (pred_v0, pred_v1)
    tbuf = (targ_v0, targ_v1)

    lane_row = lax.iota(jnp.int32, L) * ROW

    def start(s, b):
        off = base + s * SEC
        return (
            pltpu.async_copy(pred_hbm.at[pl.ds(off, SEC)], pbuf[b], psem[b]),
            pltpu.async_copy(targ_hbm.at[pl.ds(off, SEC)], tbuf[b], tsem[b]),
        )

    def chunk_fn(pred_ref, targ_ref):
      def chunk(i, acc):
        rb = lane_row + i * (L * ROW)
        gp = lambda col: plsc.load_gather(pred_ref, [rb + col])
        gt = lambda col: plsc.load_gather(targ_ref, [rb + col])
        t0, t1, t2, t3 = gt(0), gt(1), gt(2), gt(3)
        tx0, ty0, tx1, ty1 = _corners(t0, t1, t2, t3)
        ta = (tx1 - tx0) * (ty1 - ty0)
        p0, p1, p2, p3, p4 = gp(0), gp(1), gp(2), gp(3), gp(4)
        p5, p6, p7, p8, p9 = gp(5), gp(6), gp(7), gp(8), gp(9)
        iou0 = _iou(p0, p1, p2, p3, tx0, ty0, tx1, ty1, ta)
        iou1 = _iou(p5, p6, p7, p8, tx0, ty0, tx1, ty1, ta)
        sel0 = iou0 >= iou1
        max_iou = jnp.maximum(iou0, iou1)
        sx = jnp.where(sel0, p0, p5)
        sy = jnp.where(sel0, p1, p6)
        sw = jnp.where(sel0, p2, p7)
        sh = jnp.where(sel0, p3, p8)
        sc = jnp.where(sel0, p4, p9)
        ncf = jnp.where(sel0, p9, p4)
        t5, t6, t7, t8 = gt(5), gt(6), gt(7), gt(8)
        rtx = jnp.where(sel0, t0, t5)
        rty = jnp.where(sel0, t1, t6)
        rtw = jnp.where(sel0, t2, t7)
        rth = jnp.where(sel0, t3, t8)
        def sq(v):
            return v * v
        contain = sq(sc - max_iou)
        loc = (sq(sx - rtx) + sq(sy - rty)
               + sq(_sqrt(sw) - _sqrt(rtw))
               + sq(_sqrt(sh) - _sqrt(rth)))
        cls = jnp.zeros((L,), jnp.float32)
        for col in range(10, 30):
            d = gp(col) - gt(col)
            cls = cls + d * d
        t4 = gt(4)
        t9 = gt(9)
        noo = sq(p4 - t4) + sq(p9 - t9)
        coo = t4 > 0.0
        row = jnp.where(coo, 5.0 * loc + 2.0 * contain + ncf * ncf + cls,
                        0.5 * noo)
        return acc + row
      return chunk

    pending = {0: start(0, 0)}
    acc = jnp.zeros((L,), jnp.float32)
    for s in range(N_SEC):
        b = s % 2
        hp, ht = pending.pop(s)
        hp.wait()
        ht.wait()
        if s + 1 < N_SEC:
            pending[s + 1] = start(s + 1, 1 - b)
        acc = lax.fori_loop(0, SEC_CHUNKS,
                            chunk_fn(pbuf[b], tbuf[b]), acc)

    stage_v[...] = acc * np.float32(1.0 / N_BATCH)
    pltpu.sync_copy(stage_v, out_hbm.at[wid])


def kernel(pred_tensor, target_tensor):
    pf = pred_tensor.reshape(-1)
    tf = target_tensor.reshape(-1)
    sck = pl.kernel(
        _body,
        out_type=jax.ShapeDtypeStruct((N_W, L), jnp.float32),
        mesh=plsc.VectorSubcoreMesh(core_axis_name="c", subcore_axis_name="s",
                                    num_cores=NC),
        scratch_types=[
            pltpu.VMEM((SEC,), jnp.float32),
            pltpu.VMEM((SEC,), jnp.float32),
            pltpu.VMEM((SEC,), jnp.float32),
            pltpu.VMEM((SEC,), jnp.float32),
            pltpu.VMEM((L,), jnp.float32),
            pltpu.SemaphoreType.DMA,
            pltpu.SemaphoreType.DMA,
            pltpu.SemaphoreType.DMA,
            pltpu.SemaphoreType.DMA,
        ],
        compiler_params=pltpu.CompilerParams(
            needs_layout_passes=False, skip_device_barrier=True),
    )
    partials = sck(pf, tf)
    return jnp.sum(partials)


# speedup vs baseline: 1.0334x; 1.0334x over previous
import jax
import jax.numpy as jnp
import numpy as np
from jax import lax
from jax.experimental import pallas as pl
from jax.experimental.pallas import tpu as pltpu
from jax.experimental.pallas import tpu_sc as plsc

N_BATCH = 128
ROW = 30
N_ROWS = 128 * 14 * 14
NC, NS, L = 2, 16, 16
N_W = NC * NS
ROWS_PER_W = N_ROWS // N_W
CHUNKS = ROWS_PER_W // L
SLAB = ROWS_PER_W * ROW
N_SEC = 7
SEC_CHUNKS = CHUNKS // N_SEC
SEC = SEC_CHUNKS * L * ROW

INV14 = np.float32(1.0 / 14.0)


def _sqrt(x):
    i = lax.bitcast_convert_type(x, jnp.int32)
    i = 0x5F3759DF - jnp.right_shift(i, 1)
    y = lax.bitcast_convert_type(i, jnp.float32)
    for _ in range(3):
        y = y * (1.5 - 0.5 * x * y * y)
    return x * y


def _corners(x, y, w, h):
    cx = x * INV14
    cy = y * INV14
    return cx - 0.5 * w, cy - 0.5 * h, cx + 0.5 * w, cy + 0.5 * h


def _iou(px, py, pw, ph, tx0, ty0, tx1, ty1, ta):
    ax0, ay0, ax1, ay1 = _corners(px, py, pw, ph)
    iw = jnp.maximum(jnp.minimum(ax1, tx1) - jnp.maximum(ax0, tx0), 0.0)
    ih = jnp.maximum(jnp.minimum(ay1, ty1) - jnp.maximum(ay0, ty0), 0.0)
    inter = iw * ih
    a1 = (ax1 - ax0) * (ay1 - ay0)
    return inter / (a1 + ta - inter)


def _body(pred_hbm, targ_hbm, out_hbm, pred_v, targ_v, stage_v, psem, tsem):
    cc = lax.axis_index("c")
    ss = lax.axis_index("s")
    wid = cc * NS + ss
    base = wid * SLAB
    hp = pltpu.async_copy(pred_hbm.at[pl.ds(base, SLAB)], pred_v, psem)
    ht = pltpu.async_copy(targ_hbm.at[pl.ds(base, SLAB)], targ_v, tsem)
    hp.wait()
    ht.wait()

    lane_row = lax.iota(jnp.int32, L) * ROW

    def chunk(i, acc):
        rb = lane_row + i * (L * ROW)
        gp = lambda col: plsc.load_gather(pred_v, [rb + col])
        gt = lambda col: plsc.load_gather(targ_v, [rb + col])
        t0, t1, t2, t3 = gt(0), gt(1), gt(2), gt(3)
        tx0, ty0, tx1, ty1 = _corners(t0, t1, t2, t3)
        ta = (tx1 - tx0) * (ty1 - ty0)
        p0, p1, p2, p3, p4 = gp(0), gp(1), gp(2), gp(3), gp(4)
        p5, p6, p7, p8, p9 = gp(5), gp(6), gp(7), gp(8), gp(9)
        iou0 = _iou(p0, p1, p2, p3, tx0, ty0, tx1, ty1, ta)
        iou1 = _iou(p5, p6, p7, p8, tx0, ty0, tx1, ty1, ta)
        sel0 = iou0 >= iou1
        max_iou = jnp.maximum(iou0, iou1)
        sx = jnp.where(sel0, p0, p5)
        sy = jnp.where(sel0, p1, p6)
        sw = jnp.where(sel0, p2, p7)
        sh = jnp.where(sel0, p3, p8)
        sc = jnp.where(sel0, p4, p9)
        ncf = jnp.where(sel0, p9, p4)
        t5, t6, t7, t8 = gt(5), gt(6), gt(7), gt(8)
        rtx = jnp.where(sel0, t0, t5)
        rty = jnp.where(sel0, t1, t6)
        rtw = jnp.where(sel0, t2, t7)
        rth = jnp.where(sel0, t3, t8)
        def sq(v):
            return v * v
        contain = sq(sc - max_iou)
        loc = (sq(sx - rtx) + sq(sy - rty)
               + sq(_sqrt(sw) - _sqrt(rtw))
               + sq(_sqrt(sh) - _sqrt(rth)))
        cls = jnp.zeros((L,), jnp.float32)
        for col in range(10, 30):
            d = gp(col) - gt(col)
            cls = cls + d * d
        t4 = gt(4)
        t9 = gt(9)
        noo = sq(p4 - t4) + sq(p9 - t9)
        coo = t4 > 0.0
        row = jnp.where(coo, 5.0 * loc + 2.0 * contain + ncf * ncf + cls,
                        0.5 * noo)
        return acc + row

    acc = lax.fori_loop(0, CHUNKS, chunk, jnp.zeros((L,), jnp.float32))

    stage_v[...] = acc * np.float32(1.0 / N_BATCH)
    pltpu.sync_copy(stage_v, out_hbm.at[wid])


def kernel(pred_tensor, target_tensor):
    pf = pred_tensor.reshape(-1)
    tf = target_tensor.reshape(-1)
    sck = pl.kernel(
        _body,
        out_type=jax.ShapeDtypeStruct((N_W, L), jnp.float32),
        mesh=plsc.VectorSubcoreMesh(core_axis_name="c", subcore_axis_name="s",
                                    num_cores=NC),
        scratch_types=[
            pltpu.VMEM((SLAB,), jnp.float32),
            pltpu.VMEM((SLAB,), jnp.float32),
            pltpu.VMEM((L,), jnp.float32),
            pltpu.SemaphoreType.DMA,
            pltpu.SemaphoreType.DMA,
        ],
        compiler_params=pltpu.CompilerParams(
            needs_layout_passes=False, skip_device_barrier=True),
    )
    partials = sck(pf, tf)
    return jnp.sum(partials)
